# Initial kernel scaffold; baseline (speedup 1.0000x reference)
import jax
import jax.numpy as jnp
from jax.experimental import pallas as pl


def kernel(weight, x):
    raise NotImplementedError("write your pallas kernel here")


# speedup vs baseline: 1.0902x; 1.0902x over previous
import functools

import jax
import jax.numpy as jnp
from jax import lax
from jax.experimental import pallas as pl
from jax.experimental.pallas import tpu as pltpu
from jax.experimental.pallas import tpu_sc as plsc

V = 1_000_000
D = 32
N_LEVELS = 127.0

_RW = 128
_RROWS = V * D // _RW
_RB = 25_000
_GRID_R = _RROWS // _RB


def _scale_body(w_ref, scale_ref, acc_ref):
    i = pl.program_id(0)
    m = jnp.max(jnp.abs(w_ref[...]))

    @pl.when(i == 0)
    def _():
        acc_ref[0] = m

    @pl.when(i > 0)
    def _():
        acc_ref[0] = jnp.maximum(acc_ref[0], m)

    @pl.when(i == _GRID_R - 1)
    def _():
        scale_ref[0] = jnp.maximum(acc_ref[0], 1e-8) / N_LEVELS


_scale_call = pl.pallas_call(
    _scale_body,
    grid=(_GRID_R,),
    in_specs=[pl.BlockSpec((_RB, _RW), lambda i: (i, 0))],
    out_specs=pl.BlockSpec(memory_space=pltpu.SMEM),
    out_shape=jax.ShapeDtypeStruct((1,), jnp.float32),
    scratch_shapes=[pltpu.SMEM((1,), jnp.float32)],
)

_NC, _NS = 2, 16
_NW = _NC * _NS
_B = 16384 * 20
_BPW = _B // _NW
_CH = 128
_K = 8
_GCH = _CH * _K
_NG = _BPW // _GCH


def _gather_body(x_ref, w_ref, out_ref, idx_v, rows_v, semg, semw):
    c = lax.axis_index("c")
    s = lax.axis_index("s")
    wid = s * _NC + c
    base = wid * _BPW
    pltpu.sync_copy(x_ref.at[pl.ds(base, _BPW)], idx_v)
    writes = [None, None]
    for g in range(_NG):
        p = g % 2
        if writes[p] is not None:
            writes[p].wait()
        descs = [
            pltpu.async_copy(
                w_ref.at[idx_v.at[pl.ds(g * _GCH + j * _CH, _CH)]],
                rows_v.at[p, pl.ds(j * _CH, _CH)],
                semg,
            )
            for j in range(_K)
        ]
        for d_ in descs:
            d_.wait()
        writes[p] = pltpu.async_copy(
            rows_v.at[p], out_ref.at[pl.ds(base + g * _GCH, _GCH)], semw
        )
    for wdesc in writes:
        if wdesc is not None:
            wdesc.wait()


_gather_call = functools.partial(
    pl.kernel,
    mesh=plsc.VectorSubcoreMesh(
        core_axis_name="c", subcore_axis_name="s", num_cores=_NC, num_subcores=_NS
    ),
    out_type=jax.ShapeDtypeStruct((_B, D), jnp.float32),
    scratch_types=[
        pltpu.VMEM((_BPW,), jnp.int32),
        pltpu.VMEM((2, _GCH, D), jnp.float32),
        pltpu.SemaphoreType.DMA,
        pltpu.SemaphoreType.DMA,
    ],
    compiler_params=pltpu.CompilerParams(use_tc_tiling_on_sc=False),
)(_gather_body)

_QROWS = _B * D // _RW
_QB = 4096
_GRID_Q = _QROWS // _QB


def _quant_body(scale_ref, g_ref, o_ref):
    sc = scale_ref[0]
    q = jnp.clip(jnp.round(g_ref[...] / sc), -N_LEVELS, N_LEVELS - 1.0)
    o_ref[...] = q * sc


_quant_call = pl.pallas_call(
    _quant_body,
    grid=(_GRID_Q,),
    in_specs=[
        pl.BlockSpec(memory_space=pltpu.SMEM),
        pl.BlockSpec((_QB, _RW), lambda i: (i, 0)),
    ],
    out_specs=pl.BlockSpec((_QB, _RW), lambda i: (i, 0)),
    out_shape=jax.ShapeDtypeStruct((_QROWS, _RW), jnp.float32),
)


def kernel(weight, x):
    scale = _scale_call(weight.reshape(_RROWS, _RW))
    xf = x.reshape(-1)
    gathered = _gather_call(xf, weight)
    out = _quant_call(scale, gathered.reshape(_QROWS, _RW))
    return out.reshape(x.shape + (D,)), scale
